# baseline (speedup 1.0000x reference)
import functools

import jax
import jax.numpy as jnp
from jax import lax
from jax.experimental import pallas as pl
from jax.experimental.pallas import tpu as pltpu
from jax.experimental.pallas import tpu_sc as plsc

N = 10000
NPAD = 10240
BLK = 512
GRID = NPAD // BLK
HID = 256

_NC, _NS = 2, 16
_NW = _NC * _NS
_NPT = NPAD // _NW
E_ = 320000
EPAD = E_ + 512


def _rb(i):
    return (i, 0)


def _full(i):
    return (0, 0)


def _rowblock(body, n_row_in, n_full_in, out_shapes):
    def call(*args):
        in_specs = []
        for a in args[:n_row_in]:
            in_specs.append(pl.BlockSpec((BLK, a.shape[1]), _rb))
        for a in args[n_row_in:]:
            in_specs.append(pl.BlockSpec(a.shape, _full))
        out_specs = [pl.BlockSpec((BLK, s[1]), _rb) for s in out_shapes]
        out_shape = [jax.ShapeDtypeStruct(s, jnp.float32) for s in out_shapes]
        if len(out_shapes) == 1:
            out_specs, out_shape = out_specs[0], out_shape[0]
        return pl.pallas_call(
            body,
            grid=(GRID,),
            in_specs=in_specs,
            out_specs=out_specs,
            out_shape=out_shape,
            compiler_params=pltpu.CompilerParams(
                dimension_semantics=("parallel",),
            ),
        )(*args)

    return call


def _dot(a, b):
    return jax.lax.dot_general(a, b, (((1,), (0,)), ((), ())),
                               preferred_element_type=jnp.float32)


def _enc_body(x, bc, deg, A0, a0, A1, a1, B0x, B0b, b0, B1, b1,
              h0s_o, hb0s_o, dis_o):
    d = deg[...]
    dis = jnp.where(d > 0, jax.lax.rsqrt(jnp.where(d > 0, d, 1.0)), 0.0)
    dis_o[...] = dis
    hA = _dot(jnp.maximum(_dot(x[...], A0[...]) + a0[...], 0.0), A1[...]) + a1[...]
    hB = _dot(jnp.maximum(_dot(x[...], B0x[...]) + _dot(bc[...], B0b[...]) + b0[...], 0.0),
              B1[...]) + b1[...]
    h0s_o[...] = hA * dis
    hb0s_o[...] = hB * dis


def _g1_body(aggA, aggB, dis, W, b, zA_o, zB_o):
    d = dis[...]
    zA = jnp.maximum(_dot(aggA[...] * d, W[...]) + b[...], 0.0)
    zB = jnp.maximum(_dot(aggB[...] * d, W[...]) + b[...], 0.0)
    zA_o[...] = zA * d
    zB_o[...] = zB * d


def _ln(z, g, b):
    mu = jnp.mean(z, axis=-1, keepdims=True)
    var = jnp.mean((z - mu) ** 2, axis=-1, keepdims=True)
    return (z - mu) * jax.lax.rsqrt(var + 1e-5) * g + b


def _g2_body(aggA, aggB, dis, W, b, lg, lb, h_o, hs_o):
    d = dis[...]
    tA = _dot(aggA[...] * d, W[...]) + b[...]
    tB = _dot(aggB[...] * d, W[...]) + b[...]
    h = _ln(tA, lg[...], lb[...]) - _ln(tB, lg[...], lb[...])
    h_o[...] = h
    hs_o[...] = h * d


def _pro_body(agg, dis, hprev, W, b, h_o, hs_o, *, relu):
    d = dis[...]
    z = _dot(agg[...] * d, W[...]) + b[...]
    if relu:
        z = jnp.maximum(z, 0.0)
    h = z + hprev[...]
    h_o[...] = h
    hs_o[...] = h * d


def _dec_body(h, W0, b0, W1, b1, out_o):
    z = jnp.maximum(_dot(h[...], W0[...]) + b0[...], 0.0)
    out_o[...] = _dot(z, W1[...]) + b1[...]


_SC_MESH = plsc.VectorSubcoreMesh(core_axis_name="c", subcore_axis_name="s")

_CH = 16
_NB = 2
_OCT = _CH * _NB


def _edge_chunk(acc, eb, rows, cofs, c, b0, b1, n0, iota, hidk):
    dst16 = eb[1, pl.ds(cofs, 16)]
    w16 = plsc.bitcast(eb[2, pl.ds(cofs, 16)], jnp.float32)
    eidx = c + iota
    valid = (eidx >= b0) & (eidx < b1)
    w16 = jnp.where(valid, w16, 0.0)
    base16 = jnp.clip(dst16 - n0, 0, _NPT - 1) * (hidk * 16)
    onehot0 = jnp.where(iota == 0, 1.0, 0.0)
    for j in range(_CH):
        base = base16[j]
        wsc = w16[j]
        for k in range(hidk):
            contrib = (rows[j, pl.ds(k * 16, 16)] if rows is not None
                       else onehot0) * wsc
            plsc.addupdate(acc.at[pl.ds(base + k * 16, 16)], contrib)


def _sc_body(zt, edata, brows, zinit, out, acc, eb0, eb1, rowbufs, rowb,
             sems, hidk):
    wid = lax.axis_index("s") * _NC + lax.axis_index("c")
    n0 = wid * _NPT
    pltpu.sync_copy(zinit, acc)
    pltpu.sync_copy(brows.at[wid], rowb)
    rb = rowb[...]
    b0 = rb[0]
    b1 = rb[1]
    c0 = jnp.bitwise_and(b0, jnp.int32(-_OCT))
    noct = (b1 - c0 + (_OCT - 1)) // _OCT
    noct = jnp.bitwise_and(noct + 1, jnp.int32(-2))
    ob0 = c0 // _OCT
    iota = lax.iota(jnp.int32, 16)
    ebs = (eb0, eb1)

    def gathers(q):
        if zt is None:
            return
        for b in range(_NB):
            pltpu.async_copy(
                zt.at[ebs[q].at[0, pl.ds(b * _CH, _CH)]],
                rowbufs[q * _NB + b], sems[q * _NB + b])

    def gwait(q, b):
        if zt is None:
            return
        i = q * _NB + b
        pltpu.make_async_copy(zt.at[ebs[q].at[0, pl.ds(b * _CH, _CH)]],
                              rowbufs[i], sems[i]).wait()

    pltpu.sync_copy(edata.at[ob0], eb0)
    pltpu.sync_copy(edata.at[ob0 + 1], eb1)
    gathers(0)
    gathers(1)

    def pair(p, carry):
        for q in range(2):
            oi = 2 * p + q
            cbase = c0 + oi * _OCT
            for b in range(_NB):
                gwait(q, b)
                _edge_chunk(
                    acc, ebs[q],
                    rowbufs[q * _NB + b] if zt is not None else None,
                    b * _CH, cbase + b * _CH, b0, b1, n0, iota, hidk)
            pltpu.sync_copy(edata.at[ob0 + oi + 2], ebs[q])
            gathers(q)
        return carry

    lax.fori_loop(0, noct // 2, pair, 0)
    for q in range(2):
        for b in range(_NB):
            gwait(q, b)
    pltpu.sync_copy(acc, out.at[pl.ds(n0 * (hidk * 16), _NPT * (hidk * 16))])


_ROWBUFS = [pltpu.VMEM((_CH, HID), jnp.float32) for _ in range(_NB * 2)]
_SEMS = [pltpu.SemaphoreType.DMA for _ in range(_NB * 2)]


@functools.partial(
    pl.kernel,
    out_type=jax.ShapeDtypeStruct((NPAD * 16,), jnp.float32),
    mesh=_SC_MESH,
    compiler_params=pltpu.CompilerParams(needs_layout_passes=False),
    scratch_types=[
        pltpu.VMEM((_NPT * 16,), jnp.float32),
        pltpu.VMEM((3, _OCT), jnp.int32),
        pltpu.VMEM((3, _OCT), jnp.int32),
        pltpu.VMEM((16,), jnp.int32),
    ],
)
def _sc_deg(edata, brows, zinit, out, acc, eb0, eb1, rowb):
    _sc_body(None, edata, brows, zinit, out, acc, eb0, eb1, None, rowb,
             None, 1)


@functools.partial(
    pl.kernel,
    out_type=jax.ShapeDtypeStruct((NPAD * HID,), jnp.float32),
    mesh=_SC_MESH,
    compiler_params=pltpu.CompilerParams(needs_layout_passes=False),
    scratch_types=[
        pltpu.VMEM((_NPT * HID,), jnp.float32),
        pltpu.VMEM((3, _OCT), jnp.int32),
        pltpu.VMEM((3, _OCT), jnp.int32),
        pltpu.VMEM((16,), jnp.int32),
    ] + _ROWBUFS + _SEMS,
)
def _sc_conv(zt, edata, brows, zinit, out, acc, eb0, eb1, rowb, *rest):
    rowbufs = rest[:_NB * 2]
    sems = rest[_NB * 2:]
    _sc_body(zt, edata, brows, zinit, out, acc, eb0, eb1, rowbufs, rowb,
             sems, HID // 16)


def kernel(x, bc, edge_index, edge_weight, encA_W0, encA_b0, encA_W1, encA_b1,
           encB_W0, encB_b0, encB_W1, encB_b1, encG_W0, encG_b0, encG_W1,
           encG_b1, ln_g, ln_b, pro_W0, pro_b0, pro_W1, pro_b1, pro_W2, pro_b2,
           pro_W3, pro_b3, dec_W0, dec_b0, dec_W1, dec_b1):
    f32 = jnp.float32
    i32 = jnp.int32
    src = edge_index[0]
    dst = edge_index[1]
    w = edge_weight[:, -1]

    xp = jnp.zeros((NPAD, 128), f32).at[:N].set(x)
    bcp = jnp.zeros((NPAD, 16), f32).at[:N].set(bc)

    order = jnp.argsort(dst)
    src_s = jnp.zeros((EPAD,), i32).at[:E_].set(src[order])
    dst_s = jnp.zeros((EPAD,), i32).at[:E_].set(dst[order])
    w_s = jnp.zeros((EPAD,), i32).at[:E_].set(
        jax.lax.bitcast_convert_type(w[order], i32))
    edata = jnp.stack([src_s, dst_s, w_s], axis=0).reshape(
        3, EPAD // _OCT, _OCT).transpose(1, 0, 2)
    bnd = jnp.searchsorted(dst_s[:E_], jnp.arange(_NW + 1, dtype=i32) * _NPT
                           ).astype(i32)
    brows = jnp.zeros((_NW, 16), i32).at[:, 0].set(bnd[:_NW]).at[:, 1].set(
        bnd[1:])
    zinit = jnp.zeros((_NPT * HID,), f32)

    def _conv_raw(zs):
        return _sc_conv(zs, edata, brows, zinit).reshape(NPAD, HID)

    deg = _sc_deg(edata, brows, jnp.zeros((_NPT * 16,), f32)).reshape(
        NPAD, 16)[:, :1]

    r1 = lambda v: v.reshape(1, -1)
    B0x, B0b = encB_W0[:128], encB_W0[128:]
    dec_W1p = jnp.zeros((HID, 128), f32).at[:, :3].set(dec_W1)
    dec_b1p = jnp.zeros((1, 128), f32).at[0, :3].set(dec_b1)

    enc = _rowblock(_enc_body, 3, 9, [(NPAD, HID), (NPAD, HID), (NPAD, 1)])
    h0s, hb0s, dis = enc(xp, bcp, deg, encA_W0, r1(encA_b0), encA_W1,
                         r1(encA_b1), B0x, B0b, r1(encB_b0), encB_W1,
                         r1(encB_b1))

    g1 = _rowblock(_g1_body, 3, 2, [(NPAD, HID), (NPAD, HID)])
    aggA = _conv_raw(h0s)
    aggB = _conv_raw(hb0s)
    zAs, zBs = g1(aggA, aggB, dis, encG_W0, r1(encG_b0))

    g2 = _rowblock(_g2_body, 3, 4, [(NPAD, HID), (NPAD, HID)])
    aggA2 = _conv_raw(zAs)
    aggB2 = _conv_raw(zBs)
    h, hs = g2(aggA2, aggB2, dis, encG_W1, r1(encG_b1), r1(ln_g), r1(ln_b))

    pros = [(pro_W0, pro_b0), (pro_W1, pro_b1), (pro_W2, pro_b2),
            (pro_W3, pro_b3)]
    for i, (W, b) in enumerate(pros):
        agg = _conv_raw(hs)
        body = functools.partial(_pro_body, relu=(i < 3))
        h, hs = _rowblock(body, 3, 2, [(NPAD, HID), (NPAD, HID)])(
            agg, dis, h, W, r1(b))

    dec = _rowblock(_dec_body, 1, 4, [(NPAD, 128)])
    outp = dec(h, dec_W0, r1(dec_b0), dec_W1p, dec_b1p)
    return outp[:N, :3]


# speedup vs baseline: 1.8562x; 1.8562x over previous
import functools

import jax
import jax.numpy as jnp
from jax import lax
from jax.experimental import pallas as pl
from jax.experimental.pallas import tpu as pltpu
from jax.experimental.pallas import tpu_sc as plsc

N = 10000
NPAD = 10240
BLK = 512
GRID = NPAD // BLK
HID = 256

_NC, _NS = 2, 16
_NW = _NC * _NS
_NPT = NPAD // _NW
E_ = 320000
EPAD = E_ + 512


def _rb(i):
    return (i, 0)


def _full(i):
    return (0, 0)


def _rowblock(body, n_row_in, n_full_in, out_shapes):
    def call(*args):
        in_specs = []
        for a in args[:n_row_in]:
            in_specs.append(pl.BlockSpec((BLK, a.shape[1]), _rb))
        for a in args[n_row_in:]:
            in_specs.append(pl.BlockSpec(a.shape, _full))
        out_specs = [pl.BlockSpec((BLK, s[1]), _rb) for s in out_shapes]
        out_shape = [jax.ShapeDtypeStruct(s, jnp.float32) for s in out_shapes]
        if len(out_shapes) == 1:
            out_specs, out_shape = out_specs[0], out_shape[0]
        return pl.pallas_call(
            body,
            grid=(GRID,),
            in_specs=in_specs,
            out_specs=out_specs,
            out_shape=out_shape,
            compiler_params=pltpu.CompilerParams(
                dimension_semantics=("parallel",),
            ),
        )(*args)

    return call


def _dot(a, b):
    return jax.lax.dot_general(a, b, (((1,), (0,)), ((), ())),
                               preferred_element_type=jnp.float32)


def _enc_body(x, bc, deg, A0, a0, A1, a1, B0x, B0b, b0, B1, b1,
              h0s_o, hb0s_o, dis_o):
    d = deg[...]
    dis = jnp.where(d > 0, jax.lax.rsqrt(jnp.where(d > 0, d, 1.0)), 0.0)
    dis_o[...] = dis
    hA = _dot(jnp.maximum(_dot(x[...], A0[...]) + a0[...], 0.0), A1[...]) + a1[...]
    hB = _dot(jnp.maximum(_dot(x[...], B0x[...]) + _dot(bc[...], B0b[...]) + b0[...], 0.0),
              B1[...]) + b1[...]
    h0s_o[...] = hA * dis
    hb0s_o[...] = hB * dis


def _g1_body(aggA, aggB, dis, W, b, zA_o, zB_o):
    d = dis[...]
    zA = jnp.maximum(_dot(aggA[...] * d, W[...]) + b[...], 0.0)
    zB = jnp.maximum(_dot(aggB[...] * d, W[...]) + b[...], 0.0)
    zA_o[...] = zA * d
    zB_o[...] = zB * d


def _ln(z, g, b):
    mu = jnp.mean(z, axis=-1, keepdims=True)
    var = jnp.mean((z - mu) ** 2, axis=-1, keepdims=True)
    return (z - mu) * jax.lax.rsqrt(var + 1e-5) * g + b


def _g2_body(aggA, aggB, dis, W, b, lg, lb, h_o, hs_o):
    d = dis[...]
    tA = _dot(aggA[...] * d, W[...]) + b[...]
    tB = _dot(aggB[...] * d, W[...]) + b[...]
    h = _ln(tA, lg[...], lb[...]) - _ln(tB, lg[...], lb[...])
    h_o[...] = h
    hs_o[...] = h * d


def _pro_body(agg, dis, hprev, W, b, h_o, hs_o, *, relu):
    d = dis[...]
    z = _dot(agg[...] * d, W[...]) + b[...]
    if relu:
        z = jnp.maximum(z, 0.0)
    h = z + hprev[...]
    h_o[...] = h
    hs_o[...] = h * d


def _dec_body(h, W0, b0, W1, b1, out_o):
    z = jnp.maximum(_dot(h[...], W0[...]) + b0[...], 0.0)
    out_o[...] = _dot(z, W1[...]) + b1[...]


_SC_MESH = plsc.VectorSubcoreMesh(core_axis_name="c", subcore_axis_name="s")

_CH = 16
_NB = 2
_OCT = _CH * _NB


def _run_flush(acc, cur_base, a, hidk):
    for k in range(hidk):
        plsc.addupdate(acc.at[pl.ds(cur_base + k * 16, 16)], a[k])


def _edge_chunk(acc, eb, rows, cofs, c, b0, b1, n0, iota, cur, a, hidk):
    dst16 = eb[1, pl.ds(cofs, 16)]
    w16 = plsc.bitcast(eb[2, pl.ds(cofs, 16)], jnp.float32)
    eidx = c + iota
    valid = (eidx >= b0) & (eidx < b1)
    w16 = jnp.where(valid, w16, 0.0)
    base16 = jnp.clip(dst16 - n0, 0, _NPT - 1) * (hidk * 16)
    onehot0 = jnp.where(iota == 0, 1.0, 0.0)
    bases = [base16[j] for j in range(_CH)]
    wss = [w16[j] for j in range(_CH)]
    for j in range(_CH):
        nb = bases[j]
        is_new = nb != cur

        @pl.when(is_new)
        def _():
            _run_flush(acc, cur, a, hidk)

        gate = jnp.where(is_new, 0.0, 1.0)
        na = []
        for k in range(hidk):
            contrib = (rows[j, pl.ds(k * 16, 16)] if rows is not None
                       else onehot0) * wss[j]
            na.append(a[k] * gate + contrib)
        a = na
        cur = nb
    return cur, a


def _sc_body(zt, edata, brows, zinit, out, acc, eb0, eb1, rowbufs, rowb,
             sems, hidk):
    wid = lax.axis_index("s") * _NC + lax.axis_index("c")
    n0 = wid * _NPT
    pltpu.sync_copy(zinit, acc)
    pltpu.sync_copy(brows.at[wid], rowb)
    rb = rowb[...]
    b0 = rb[0]
    b1 = rb[1]
    c0 = jnp.bitwise_and(b0, jnp.int32(-_OCT))
    noct = (b1 - c0 + (_OCT - 1)) // _OCT
    noct = jnp.bitwise_and(noct + 1, jnp.int32(-2))
    ob0 = c0 // _OCT
    iota = lax.iota(jnp.int32, 16)
    ebs = (eb0, eb1)

    def gathers(q):
        if zt is None:
            return
        for b in range(_NB):
            pltpu.async_copy(
                zt.at[ebs[q].at[0, pl.ds(b * _CH, _CH)]],
                rowbufs[q * _NB + b], sems[q * _NB + b])

    def gwait(q, b):
        if zt is None:
            return
        i = q * _NB + b
        pltpu.make_async_copy(zt.at[ebs[q].at[0, pl.ds(b * _CH, _CH)]],
                              rowbufs[i], sems[i]).wait()

    pltpu.sync_copy(edata.at[ob0], eb0)
    pltpu.sync_copy(edata.at[ob0 + 1], eb1)
    gathers(0)
    gathers(1)

    def pair(p, carry):
        cur, a = carry
        for q in range(2):
            oi = 2 * p + q
            cbase = c0 + oi * _OCT
            for b in range(_NB):
                gwait(q, b)
                cur, a = _edge_chunk(
                    acc, ebs[q],
                    rowbufs[q * _NB + b] if zt is not None else None,
                    b * _CH, cbase + b * _CH, b0, b1, n0, iota, cur, a, hidk)
            pltpu.sync_copy(edata.at[ob0 + oi + 2], ebs[q])
            gathers(q)
        return cur, a

    a0 = [jnp.zeros((16,), jnp.float32)] * hidk
    cur, a = lax.fori_loop(0, noct // 2, pair, (jnp.int32(0), a0))
    for q in range(2):
        for b in range(_NB):
            gwait(q, b)
    _run_flush(acc, cur, a, hidk)
    pltpu.sync_copy(acc, out.at[pl.ds(n0 * (hidk * 16), _NPT * (hidk * 16))])


_ROWBUFS = [pltpu.VMEM((_CH, HID), jnp.float32) for _ in range(_NB * 2)]
_SEMS = [pltpu.SemaphoreType.DMA for _ in range(_NB * 2)]


@functools.partial(
    pl.kernel,
    out_type=jax.ShapeDtypeStruct((NPAD * 16,), jnp.float32),
    mesh=_SC_MESH,
    compiler_params=pltpu.CompilerParams(needs_layout_passes=False),
    scratch_types=[
        pltpu.VMEM((_NPT * 16,), jnp.float32),
        pltpu.VMEM((3, _OCT), jnp.int32),
        pltpu.VMEM((3, _OCT), jnp.int32),
        pltpu.VMEM((16,), jnp.int32),
    ],
)
def _sc_deg(edata, brows, zinit, out, acc, eb0, eb1, rowb):
    _sc_body(None, edata, brows, zinit, out, acc, eb0, eb1, None, rowb,
             None, 1)


@functools.partial(
    pl.kernel,
    out_type=jax.ShapeDtypeStruct((NPAD * HID,), jnp.float32),
    mesh=_SC_MESH,
    compiler_params=pltpu.CompilerParams(needs_layout_passes=False),
    scratch_types=[
        pltpu.VMEM((_NPT * HID,), jnp.float32),
        pltpu.VMEM((3, _OCT), jnp.int32),
        pltpu.VMEM((3, _OCT), jnp.int32),
        pltpu.VMEM((16,), jnp.int32),
    ] + _ROWBUFS + _SEMS,
)
def _sc_conv(zt, edata, brows, zinit, out, acc, eb0, eb1, rowb, *rest):
    rowbufs = rest[:_NB * 2]
    sems = rest[_NB * 2:]
    _sc_body(zt, edata, brows, zinit, out, acc, eb0, eb1, rowbufs, rowb,
             sems, HID // 16)


def kernel(x, bc, edge_index, edge_weight, encA_W0, encA_b0, encA_W1, encA_b1,
           encB_W0, encB_b0, encB_W1, encB_b1, encG_W0, encG_b0, encG_W1,
           encG_b1, ln_g, ln_b, pro_W0, pro_b0, pro_W1, pro_b1, pro_W2, pro_b2,
           pro_W3, pro_b3, dec_W0, dec_b0, dec_W1, dec_b1):
    f32 = jnp.float32
    i32 = jnp.int32
    src = edge_index[0]
    dst = edge_index[1]
    w = edge_weight[:, -1]

    xp = jnp.zeros((NPAD, 128), f32).at[:N].set(x)
    bcp = jnp.zeros((NPAD, 16), f32).at[:N].set(bc)

    order = jnp.argsort(dst)
    src_s = jnp.zeros((EPAD,), i32).at[:E_].set(src[order])
    dst_s = jnp.zeros((EPAD,), i32).at[:E_].set(dst[order])
    w_s = jnp.zeros((EPAD,), i32).at[:E_].set(
        jax.lax.bitcast_convert_type(w[order], i32))
    edata = jnp.stack([src_s, dst_s, w_s], axis=0).reshape(
        3, EPAD // _OCT, _OCT).transpose(1, 0, 2)
    bnd = jnp.searchsorted(dst_s[:E_], jnp.arange(_NW + 1, dtype=i32) * _NPT
                           ).astype(i32)
    brows = jnp.zeros((_NW, 16), i32).at[:, 0].set(bnd[:_NW]).at[:, 1].set(
        bnd[1:])
    zinit = jnp.zeros((_NPT * HID,), f32)

    def _conv_raw(zs):
        return _sc_conv(zs, edata, brows, zinit).reshape(NPAD, HID)

    deg = _sc_deg(edata, brows, jnp.zeros((_NPT * 16,), f32)).reshape(
        NPAD, 16)[:, :1]

    r1 = lambda v: v.reshape(1, -1)
    B0x, B0b = encB_W0[:128], encB_W0[128:]
    dec_W1p = jnp.zeros((HID, 128), f32).at[:, :3].set(dec_W1)
    dec_b1p = jnp.zeros((1, 128), f32).at[0, :3].set(dec_b1)

    enc = _rowblock(_enc_body, 3, 9, [(NPAD, HID), (NPAD, HID), (NPAD, 1)])
    h0s, hb0s, dis = enc(xp, bcp, deg, encA_W0, r1(encA_b0), encA_W1,
                         r1(encA_b1), B0x, B0b, r1(encB_b0), encB_W1,
                         r1(encB_b1))

    g1 = _rowblock(_g1_body, 3, 2, [(NPAD, HID), (NPAD, HID)])
    aggA = _conv_raw(h0s)
    aggB = _conv_raw(hb0s)
    zAs, zBs = g1(aggA, aggB, dis, encG_W0, r1(encG_b0))

    g2 = _rowblock(_g2_body, 3, 4, [(NPAD, HID), (NPAD, HID)])
    aggA2 = _conv_raw(zAs)
    aggB2 = _conv_raw(zBs)
    h, hs = g2(aggA2, aggB2, dis, encG_W1, r1(encG_b1), r1(ln_g), r1(ln_b))

    pros = [(pro_W0, pro_b0), (pro_W1, pro_b1), (pro_W2, pro_b2),
            (pro_W3, pro_b3)]
    for i, (W, b) in enumerate(pros):
        agg = _conv_raw(hs)
        body = functools.partial(_pro_body, relu=(i < 3))
        h, hs = _rowblock(body, 3, 2, [(NPAD, HID), (NPAD, HID)])(
            agg, dis, h, W, r1(b))

    dec = _rowblock(_dec_body, 1, 4, [(NPAD, 128)])
    outp = dec(h, dec_W0, r1(dec_b0), dec_W1p, dec_b1p)
    return outp[:N, :3]


# speedup vs baseline: 1.9982x; 1.0765x over previous
import functools

import jax
import jax.numpy as jnp
from jax import lax
from jax.experimental import pallas as pl
from jax.experimental.pallas import tpu as pltpu
from jax.experimental.pallas import tpu_sc as plsc

N = 10000
NPAD = 10240
BLK = 512
GRID = NPAD // BLK
HID = 256

_NC, _NS = 2, 16
_NW = _NC * _NS
_NPT = NPAD // _NW
E_ = 320000
EPAD = E_ + 512


def _rb(i):
    return (i, 0)


def _full(i):
    return (0, 0)


def _rowblock(body, n_row_in, n_full_in, out_shapes):
    def call(*args):
        in_specs = []
        for a in args[:n_row_in]:
            in_specs.append(pl.BlockSpec((BLK, a.shape[1]), _rb))
        for a in args[n_row_in:]:
            in_specs.append(pl.BlockSpec(a.shape, _full))
        out_specs = [pl.BlockSpec((BLK, s[1]), _rb) for s in out_shapes]
        out_shape = [jax.ShapeDtypeStruct(s, jnp.float32) for s in out_shapes]
        if len(out_shapes) == 1:
            out_specs, out_shape = out_specs[0], out_shape[0]
        return pl.pallas_call(
            body,
            grid=(GRID,),
            in_specs=in_specs,
            out_specs=out_specs,
            out_shape=out_shape,
            compiler_params=pltpu.CompilerParams(
                dimension_semantics=("parallel",),
            ),
        )(*args)

    return call


def _dot(a, b):
    return jax.lax.dot_general(a, b, (((1,), (0,)), ((), ())),
                               preferred_element_type=jnp.float32)


def _enc_body(x, bc, deg, A0, a0, A1, a1, B0x, B0b, b0, B1, b1,
              h0s_o, hb0s_o, dis_o):
    d = deg[...]
    dis = jnp.where(d > 0, jax.lax.rsqrt(jnp.where(d > 0, d, 1.0)), 0.0)
    dis_o[...] = dis
    hA = _dot(jnp.maximum(_dot(x[...], A0[...]) + a0[...], 0.0), A1[...]) + a1[...]
    hB = _dot(jnp.maximum(_dot(x[...], B0x[...]) + _dot(bc[...], B0b[...]) + b0[...], 0.0),
              B1[...]) + b1[...]
    h0s_o[...] = hA * dis
    hb0s_o[...] = hB * dis


def _g1_body(aggA, aggB, dis, W, b, zA_o, zB_o):
    d = dis[...]
    zA = jnp.maximum(_dot(aggA[...] * d, W[...]) + b[...], 0.0)
    zB = jnp.maximum(_dot(aggB[...] * d, W[...]) + b[...], 0.0)
    zA_o[...] = zA * d
    zB_o[...] = zB * d


def _ln(z, g, b):
    mu = jnp.mean(z, axis=-1, keepdims=True)
    var = jnp.mean((z - mu) ** 2, axis=-1, keepdims=True)
    return (z - mu) * jax.lax.rsqrt(var + 1e-5) * g + b


def _g2_body(aggA, aggB, dis, W, b, lg, lb, h_o, hs_o):
    d = dis[...]
    tA = _dot(aggA[...] * d, W[...]) + b[...]
    tB = _dot(aggB[...] * d, W[...]) + b[...]
    h = _ln(tA, lg[...], lb[...]) - _ln(tB, lg[...], lb[...])
    h_o[...] = h
    hs_o[...] = h * d


def _pro_body(agg, dis, hprev, W, b, h_o, hs_o, *, relu):
    d = dis[...]
    z = _dot(agg[...] * d, W[...]) + b[...]
    if relu:
        z = jnp.maximum(z, 0.0)
    h = z + hprev[...]
    h_o[...] = h
    hs_o[...] = h * d


def _dec_body(h, W0, b0, W1, b1, out_o):
    z = jnp.maximum(_dot(h[...], W0[...]) + b0[...], 0.0)
    out_o[...] = _dot(z, W1[...]) + b1[...]


_SC_MESH = plsc.VectorSubcoreMesh(core_axis_name="c", subcore_axis_name="s")

_CH = 16
_NQ = 4


def _run_flush(acc, cur_base, a, hidk):
    for k in range(hidk):
        plsc.addupdate(acc.at[pl.ds(cur_base + k * 16, 16)], a[k])


def _edge_chunk(acc, eb, rows, c, b0, b1, n0, iota, cur, a, hidk):
    dst16 = eb[1, pl.ds(0, 16)]
    w16 = plsc.bitcast(eb[2, pl.ds(0, 16)], jnp.float32)
    eidx = c + iota
    valid = (eidx >= b0) & (eidx < b1)
    w16 = jnp.where(valid, w16, 0.0)
    base16 = jnp.clip(dst16 - n0, 0, _NPT - 1) * (hidk * 16)
    onehot0 = jnp.where(iota == 0, 1.0, 0.0)
    bases = [base16[j] for j in range(_CH)]
    wss = [w16[j] for j in range(_CH)]
    for j in range(_CH):
        nb = bases[j]
        is_new = nb != cur

        @pl.when(is_new)
        def _():
            _run_flush(acc, cur, a, hidk)

        gate = jnp.where(is_new, 0.0, 1.0)
        na = []
        for k in range(hidk):
            contrib = (rows[j, pl.ds(k * 16, 16)] if rows is not None
                       else onehot0) * wss[j]
            na.append(a[k] * gate + contrib)
        a = na
        cur = nb
    return cur, a


def _sc_body(zt, edata, brows, zinit, out, acc, ebs, rowb, rowbufs, gsems,
             edsems, hidk):
    wid = lax.axis_index("s") * _NC + lax.axis_index("c")
    n0 = wid * _NPT
    pltpu.sync_copy(zinit, acc)
    pltpu.sync_copy(brows.at[wid], rowb)
    rb = rowb[...]
    b0 = rb[0]
    b1 = rb[1]
    c0 = jnp.bitwise_and(b0, jnp.int32(-_CH))
    ob0 = c0 // _CH
    nch = (b1 - c0 + (_CH - 1)) // _CH
    nr = (nch + (_NQ - 1)) // _NQ
    iota = lax.iota(jnp.int32, 16)

    def ed_start(q, blk):
        pltpu.async_copy(edata.at[blk], ebs[q], edsems[q])

    def ed_wait(q, blk):
        pltpu.make_async_copy(edata.at[blk], ebs[q], edsems[q]).wait()

    def g_start(q):
        if zt is not None:
            pltpu.async_copy(zt.at[ebs[q].at[0]], rowbufs[q], gsems[q])

    def g_wait(q):
        if zt is not None:
            pltpu.make_async_copy(zt.at[ebs[q].at[0]], rowbufs[q],
                                  gsems[q]).wait()

    for q in range(_NQ):
        pltpu.sync_copy(edata.at[ob0 + q], ebs[q])
        g_start(q)

    def rnd(r, carry):
        cur, a = carry
        nxt = ob0 + (r + 1) * _NQ
        for q in range(_NQ):
            g_wait(q)
            cur, a = _edge_chunk(
                acc, ebs[q], rowbufs[q] if zt is not None else None,
                c0 + (r * _NQ + q) * _CH, b0, b1, n0, iota, cur, a, hidk)
            ed_start(q, nxt + q)
            if q > 0:
                ed_wait(q - 1, nxt + q - 1)
                g_start(q - 1)
        ed_wait(_NQ - 1, nxt + _NQ - 1)
        g_start(_NQ - 1)
        return cur, a

    a0 = [jnp.zeros((16,), jnp.float32)] * hidk
    cur, a = lax.fori_loop(0, nr, rnd, (jnp.int32(0), a0))
    for q in range(_NQ):
        g_wait(q)
    _run_flush(acc, cur, a, hidk)
    pltpu.sync_copy(acc, out.at[pl.ds(n0 * (hidk * 16), _NPT * (hidk * 16))])


_EBS = [pltpu.VMEM((3, _CH), jnp.int32) for _ in range(_NQ)]
_ROWBUFS = [pltpu.VMEM((_CH, HID), jnp.float32) for _ in range(_NQ)]
_GSEMS = [pltpu.SemaphoreType.DMA for _ in range(_NQ)]
_EDSEMS = [pltpu.SemaphoreType.DMA for _ in range(_NQ)]


@functools.partial(
    pl.kernel,
    out_type=jax.ShapeDtypeStruct((NPAD * 16,), jnp.float32),
    mesh=_SC_MESH,
    compiler_params=pltpu.CompilerParams(needs_layout_passes=False),
    scratch_types=[pltpu.VMEM((_NPT * 16,), jnp.float32)] + _EBS
    + [pltpu.VMEM((16,), jnp.int32)] + _EDSEMS,
)
def _sc_deg(edata, brows, zinit, out, acc, *rest):
    ebs = rest[:_NQ]
    rowb = rest[_NQ]
    edsems = rest[_NQ + 1:]
    _sc_body(None, edata, brows, zinit, out, acc, ebs, rowb, None, None,
             edsems, 1)


@functools.partial(
    pl.kernel,
    out_type=jax.ShapeDtypeStruct((NPAD * HID,), jnp.float32),
    mesh=_SC_MESH,
    compiler_params=pltpu.CompilerParams(needs_layout_passes=False),
    scratch_types=[pltpu.VMEM((_NPT * HID,), jnp.float32)] + _EBS
    + [pltpu.VMEM((16,), jnp.int32)] + _ROWBUFS + _GSEMS + _EDSEMS,
)
def _sc_conv(zt, edata, brows, zinit, out, acc, *rest):
    ebs = rest[:_NQ]
    rowb = rest[_NQ]
    rowbufs = rest[_NQ + 1:2 * _NQ + 1]
    gsems = rest[2 * _NQ + 1:3 * _NQ + 1]
    edsems = rest[3 * _NQ + 1:]
    _sc_body(zt, edata, brows, zinit, out, acc, ebs, rowb, rowbufs, gsems,
             edsems, HID // 16)


def kernel(x, bc, edge_index, edge_weight, encA_W0, encA_b0, encA_W1, encA_b1,
           encB_W0, encB_b0, encB_W1, encB_b1, encG_W0, encG_b0, encG_W1,
           encG_b1, ln_g, ln_b, pro_W0, pro_b0, pro_W1, pro_b1, pro_W2, pro_b2,
           pro_W3, pro_b3, dec_W0, dec_b0, dec_W1, dec_b1):
    f32 = jnp.float32
    i32 = jnp.int32
    src = edge_index[0]
    dst = edge_index[1]
    w = edge_weight[:, -1]

    xp = jnp.zeros((NPAD, 128), f32).at[:N].set(x)
    bcp = jnp.zeros((NPAD, 16), f32).at[:N].set(bc)

    order = jnp.argsort(dst)
    src_s = jnp.zeros((EPAD,), i32).at[:E_].set(src[order])
    dst_s = jnp.zeros((EPAD,), i32).at[:E_].set(dst[order])
    w_s = jnp.zeros((EPAD,), i32).at[:E_].set(
        jax.lax.bitcast_convert_type(w[order], i32))
    edata = jnp.stack([src_s, dst_s, w_s], axis=0).reshape(
        3, EPAD // _CH, _CH).transpose(1, 0, 2)
    bnd = jnp.searchsorted(dst_s[:E_], jnp.arange(_NW + 1, dtype=i32) * _NPT
                           ).astype(i32)
    brows = jnp.zeros((_NW, 16), i32).at[:, 0].set(bnd[:_NW]).at[:, 1].set(
        bnd[1:])
    zinit = jnp.zeros((_NPT * HID,), f32)

    def _conv_raw(zs):
        return _sc_conv(zs, edata, brows, zinit).reshape(NPAD, HID)

    deg = _sc_deg(edata, brows, jnp.zeros((_NPT * 16,), f32)).reshape(
        NPAD, 16)[:, :1]

    r1 = lambda v: v.reshape(1, -1)
    B0x, B0b = encB_W0[:128], encB_W0[128:]
    dec_W1p = jnp.zeros((HID, 128), f32).at[:, :3].set(dec_W1)
    dec_b1p = jnp.zeros((1, 128), f32).at[0, :3].set(dec_b1)

    enc = _rowblock(_enc_body, 3, 9, [(NPAD, HID), (NPAD, HID), (NPAD, 1)])
    h0s, hb0s, dis = enc(xp, bcp, deg, encA_W0, r1(encA_b0), encA_W1,
                         r1(encA_b1), B0x, B0b, r1(encB_b0), encB_W1,
                         r1(encB_b1))

    g1 = _rowblock(_g1_body, 3, 2, [(NPAD, HID), (NPAD, HID)])
    aggA = _conv_raw(h0s)
    aggB = _conv_raw(hb0s)
    zAs, zBs = g1(aggA, aggB, dis, encG_W0, r1(encG_b0))

    g2 = _rowblock(_g2_body, 3, 4, [(NPAD, HID), (NPAD, HID)])
    aggA2 = _conv_raw(zAs)
    aggB2 = _conv_raw(zBs)
    h, hs = g2(aggA2, aggB2, dis, encG_W1, r1(encG_b1), r1(ln_g), r1(ln_b))

    pros = [(pro_W0, pro_b0), (pro_W1, pro_b1), (pro_W2, pro_b2),
            (pro_W3, pro_b3)]
    for i, (W, b) in enumerate(pros):
        agg = _conv_raw(hs)
        body = functools.partial(_pro_body, relu=(i < 3))
        h, hs = _rowblock(body, 3, 2, [(NPAD, HID), (NPAD, HID)])(
            agg, dis, h, W, r1(b))

    dec = _rowblock(_dec_body, 1, 4, [(NPAD, 128)])
    outp = dec(h, dec_W0, r1(dec_b0), dec_W1p, dec_b1p)
    return outp[:N, :3]
